# baseline (speedup 1.0000x reference)
import functools

import jax
import jax.numpy as jnp
from jax import lax
from jax.experimental import pallas as pl
from jax.experimental.pallas import tpu as pltpu
from jax.experimental.pallas import tpu_sc as plsc

N = 10000
E = 640000
D = 128
DE = 16
G = 64

NC = 2
NS = 16
NW = NC * NS
EPW = E // NW
C = 40
NCHUNK = EPW // C
WTILES = 10
ROWS_PER_WTILE = N // WTILES
ZROWS = 40

_LP6 = -1.72080611e-02
_LP5 = 8.17268084e-02
_LP4 = -1.88782674e-01
_LP3 = 3.14590535e-01
_LP2 = -4.96977911e-01
_LP1 = 9.99792436e-01
_LP0 = 3.50755205e-06


def _sc_edge_body(dtab, stab, etab, dsti, srci, out,
                  didx, sidx, dbuf, sbuf, ebuf, mbuf, zbuf, acc,
                  semd, sems, seme):
    cid = lax.axis_index("c")
    sid = lax.axis_index("s")
    wid = cid * NS + sid

    def _zfill(i, _):
        r = i // 8
        l = (i % 8) * 16
        zbuf[r, pl.ds(l, 16)] = jnp.zeros((16,), jnp.float32)
        return 0
    lax.fori_loop(0, ZROWS * 8, _zfill, 0)

    @pl.when(sid < WTILES)
    def _zero_acc():
        for j in range(ROWS_PER_WTILE // ZROWS):
            pltpu.sync_copy(
                zbuf, acc.at[pl.ds(sid * ROWS_PER_WTILE + j * ZROWS, ZROWS)])

    plsc.subcore_barrier()

    base_w = wid * EPW

    def _chunk(g, _):
        eb = base_w + g * C
        pltpu.sync_copy(dsti.at[pl.ds(eb, C)], didx)
        pltpu.sync_copy(srci.at[pl.ds(eb, C)], sidx)
        cpd = pltpu.async_copy(dtab.at[didx], dbuf, semd)
        cps = pltpu.async_copy(stab.at[sidx], sbuf, sems)
        cpe = pltpu.async_copy(etab.at[pl.ds(eb, C)], ebuf, seme)
        cpd.wait()
        cps.wait()
        cpe.wait()

        def _edge(e, _):
            for k in range(8):
                lo = k * 16
                f = (dbuf[e, pl.ds(lo, 16)] + sbuf[e, pl.ds(lo, 16)]
                     + ebuf[e, pl.ds(lo, 16)])
                s = (dbuf[e, pl.ds(128 + lo, 16)]
                     + sbuf[e, pl.ds(128 + lo, 16)]
                     + ebuf[e, pl.ds(128 + lo, 16)])
                sig = 1.0 / (1.0 + jnp.exp(-f))
                u = jnp.exp(-jnp.abs(s))
                p = _LP6
                p = p * u + _LP5
                p = p * u + _LP4
                p = p * u + _LP3
                p = p * u + _LP2
                p = p * u + _LP1
                p = p * u + _LP0
                sp = jnp.maximum(s, 0.0) + p
                mbuf[e, pl.ds(lo, 16)] = sig * sp
            return 0

        lax.fori_loop(0, C, _edge, 0)
        pltpu.sync_copy(mbuf, acc.at[didx], add=True)
        return 0

    lax.fori_loop(0, NCHUNK, _chunk, 0)
    plsc.subcore_barrier()

    @pl.when(sid < WTILES)
    def _writeout():
        pltpu.sync_copy(acc.at[pl.ds(sid * ROWS_PER_WTILE, ROWS_PER_WTILE)],
                        out.at[cid, pl.ds(sid * ROWS_PER_WTILE,
                                          ROWS_PER_WTILE)])


_sc_edge_pass = functools.partial(
    pl.kernel,
    out_type=jax.ShapeDtypeStruct((NC, N, D), jnp.float32),
    mesh=plsc.VectorSubcoreMesh(core_axis_name="c", subcore_axis_name="s"),
    scratch_types=[
        pltpu.VMEM((C,), jnp.int32),
        pltpu.VMEM((C,), jnp.int32),
        pltpu.VMEM((C, 2 * D), jnp.float32),
        pltpu.VMEM((C, 2 * D), jnp.float32),
        pltpu.VMEM((C, 2 * D), jnp.float32),
        pltpu.VMEM((C, D), jnp.float32),
        pltpu.VMEM((ZROWS, D), jnp.float32),
        pltpu.VMEM_SHARED((N, D), jnp.float32),
        pltpu.SemaphoreType.DMA,
        pltpu.SemaphoreType.DMA,
        pltpu.SemaphoreType.DMA,
    ],
)(_sc_edge_body)


_RB = 1000
_NG = N // _RB
_EB = 2000
_EG = E // _EB


def _tables_body(h_ref, wd_ref, ws_ref, d_ref, s_ref):
    h = h_ref[...]
    d_ref[...] = jnp.dot(h, wd_ref[...], preferred_element_type=jnp.float32)
    s_ref[...] = jnp.dot(h, ws_ref[...], preferred_element_type=jnp.float32)


def _tc_tables(h, Wd, Ws):
    return pl.pallas_call(
        _tables_body,
        grid=(_NG,),
        in_specs=[
            pl.BlockSpec((_RB, D), lambda i: (i, 0)),
            pl.BlockSpec((D, 2 * D), lambda i: (0, 0)),
            pl.BlockSpec((D, 2 * D), lambda i: (0, 0)),
        ],
        out_specs=[pl.BlockSpec((_RB, 2 * D), lambda i: (i, 0))] * 2,
        out_shape=[jax.ShapeDtypeStruct((N, 2 * D), jnp.float32)] * 2,
    )(h, Wd, Ws)


def _etab_body(ea_ref, we_ref, be_ref, o_ref):
    o_ref[...] = (jnp.dot(ea_ref[...], we_ref[...],
                          preferred_element_type=jnp.float32) + be_ref[...])


def _tc_etab(ea, We, be):
    return pl.pallas_call(
        _etab_body,
        grid=(_EG,),
        in_specs=[
            pl.BlockSpec((_EB, DE), lambda i: (i, 0)),
            pl.BlockSpec((DE, 2 * D), lambda i: (0, 0)),
            pl.BlockSpec((1, 2 * D), lambda i: (0, 0)),
        ],
        out_specs=pl.BlockSpec((_EB, 2 * D), lambda i: (i, 0)),
        out_shape=jax.ShapeDtypeStruct((E, 2 * D), jnp.float32),
    )(ea, We, be)


def _comb1_body(x_ref, a_ref, wl_ref, bl_ref, wd_ref, ws_ref,
                h_ref, d_ref, s_ref):
    hr = jnp.maximum(x_ref[...] + a_ref[0] + a_ref[1], 0.0)
    h = jnp.dot(hr, wl_ref[...], preferred_element_type=jnp.float32) + bl_ref[...]
    h_ref[...] = h
    d_ref[...] = jnp.dot(h, wd_ref[...], preferred_element_type=jnp.float32)
    s_ref[...] = jnp.dot(h, ws_ref[...], preferred_element_type=jnp.float32)


def _tc_combine_lin(x, agg, Wl, bl, Wd, Ws):
    return pl.pallas_call(
        _comb1_body,
        grid=(_NG,),
        in_specs=[
            pl.BlockSpec((_RB, D), lambda i: (i, 0)),
            pl.BlockSpec((NC, _RB, D), lambda i: (0, i, 0)),
            pl.BlockSpec((D, D), lambda i: (0, 0)),
            pl.BlockSpec((1, D), lambda i: (0, 0)),
            pl.BlockSpec((D, 2 * D), lambda i: (0, 0)),
            pl.BlockSpec((D, 2 * D), lambda i: (0, 0)),
        ],
        out_specs=[
            pl.BlockSpec((_RB, D), lambda i: (i, 0)),
            pl.BlockSpec((_RB, 2 * D), lambda i: (i, 0)),
            pl.BlockSpec((_RB, 2 * D), lambda i: (i, 0)),
        ],
        out_shape=[
            jax.ShapeDtypeStruct((N, D), jnp.float32),
            jax.ShapeDtypeStruct((N, 2 * D), jnp.float32),
            jax.ShapeDtypeStruct((N, 2 * D), jnp.float32),
        ],
    )(x, agg, Wl, bl, Wd, Ws)


def _comb2_body(x_ref, a_ref, wd_ref, ws_ref, h_ref, d_ref, s_ref):
    h = jnp.maximum(x_ref[...] + a_ref[0] + a_ref[1], 0.0)
    h_ref[...] = h
    d_ref[...] = jnp.dot(h, wd_ref[...], preferred_element_type=jnp.float32)
    s_ref[...] = jnp.dot(h, ws_ref[...], preferred_element_type=jnp.float32)


def _tc_combine(x, agg, Wd, Ws):
    return pl.pallas_call(
        _comb2_body,
        grid=(_NG,),
        in_specs=[
            pl.BlockSpec((_RB, D), lambda i: (i, 0)),
            pl.BlockSpec((NC, _RB, D), lambda i: (0, i, 0)),
            pl.BlockSpec((D, 2 * D), lambda i: (0, 0)),
            pl.BlockSpec((D, 2 * D), lambda i: (0, 0)),
        ],
        out_specs=[
            pl.BlockSpec((_RB, D), lambda i: (i, 0)),
            pl.BlockSpec((_RB, 2 * D), lambda i: (i, 0)),
            pl.BlockSpec((_RB, 2 * D), lambda i: (i, 0)),
        ],
        out_shape=[
            jax.ShapeDtypeStruct((N, D), jnp.float32),
            jax.ShapeDtypeStruct((N, 2 * D), jnp.float32),
            jax.ShapeDtypeStruct((N, 2 * D), jnp.float32),
        ],
    )(x, agg, Wd, Ws)


def _pool_body(x_ref, a_ref, b_ref, sum_ref, cnt_ref):
    i = pl.program_id(0)
    h3 = jnp.maximum(x_ref[...] + a_ref[0] + a_ref[1], 0.0)
    bids = b_ref[0, 0, :]
    oh = (lax.broadcasted_iota(jnp.int32, (G, _RB), 0)
          == bids[None, :]).astype(jnp.float32)
    ps = jnp.dot(oh, h3, preferred_element_type=jnp.float32)
    pc = jnp.broadcast_to(jnp.sum(oh, axis=1)[:, None], (G, D))

    @pl.when(i == 0)
    def _():
        sum_ref[...] = ps
        cnt_ref[...] = pc

    @pl.when(i > 0)
    def _():
        sum_ref[...] += ps
        cnt_ref[...] += pc


def _tc_pool(h2, agg, batch3d):
    return pl.pallas_call(
        _pool_body,
        grid=(_NG,),
        in_specs=[
            pl.BlockSpec((_RB, D), lambda i: (i, 0)),
            pl.BlockSpec((NC, _RB, D), lambda i: (0, i, 0)),
            pl.BlockSpec((1, 1, _RB), lambda i: (i, 0, 0)),
        ],
        out_specs=[pl.BlockSpec((G, D), lambda i: (0, 0))] * 2,
        out_shape=[jax.ShapeDtypeStruct((G, D), jnp.float32)] * 2,
    )(h2, agg, batch3d)


def _head_body(s_ref, c_ref, w1_ref, b1_ref, w2_ref, b2_ref, w3_ref, b3_ref,
               o_ref):
    g = s_ref[...] / jnp.maximum(c_ref[...], 1.0)
    g = jnp.maximum(jnp.dot(g, w1_ref[...],
                            preferred_element_type=jnp.float32) + b1_ref[...],
                    0.0)
    g = jnp.maximum(jnp.dot(g, w2_ref[...],
                            preferred_element_type=jnp.float32) + b2_ref[...],
                    0.0)
    o = jnp.sum(g * w3_ref[...], axis=1, keepdims=True) + b3_ref[0, 0]
    o_ref[...] = jnp.broadcast_to(o, (G, D))


def _tc_head(sums, cnts, Wh1, bh1, Wh2, bh2, w3row, bh3):
    return pl.pallas_call(
        _head_body,
        in_specs=[
            pl.BlockSpec((G, D), lambda: (0, 0)),
            pl.BlockSpec((G, D), lambda: (0, 0)),
            pl.BlockSpec((D, D), lambda: (0, 0)),
            pl.BlockSpec((1, D), lambda: (0, 0)),
            pl.BlockSpec((D, D), lambda: (0, 0)),
            pl.BlockSpec((1, D), lambda: (0, 0)),
            pl.BlockSpec((1, D), lambda: (0, 0)),
            pl.BlockSpec((1, D), lambda: (0, 0)),
        ],
        out_specs=pl.BlockSpec((G, D), lambda: (0, 0)),
        out_shape=jax.ShapeDtypeStruct((G, D), jnp.float32),
    )(sums, cnts, Wh1, bh1, Wh2, bh2, w3row, bh3)


def _split_w(Wf, Ws):
    Wd = jnp.concatenate([Wf[:D], Ws[:D]], axis=1)
    Wsrc = jnp.concatenate([Wf[D:2 * D], Ws[D:2 * D]], axis=1)
    We = jnp.concatenate([Wf[2 * D:], Ws[2 * D:]], axis=1)
    return Wd, Wsrc, We


def kernel(x, edge_index, edge_attr, batch,
           Wf1, bf1, Ws1, bs1, Wl, bl,
           Wf2, bf2, Ws2, bs2, Wf3, bf3, Ws3, bs3,
           Wh1, bh1, Wh2, bh2, Wh3, bh3):
    src = edge_index[0].astype(jnp.int32)
    dst = edge_index[1].astype(jnp.int32)
    batch3d = batch.astype(jnp.int32).reshape(_NG, 1, _RB)

    Wd1, Wsr1, We1 = _split_w(Wf1, Ws1)
    Wd2, Wsr2, We2 = _split_w(Wf2, Ws2)
    Wd3, Wsr3, We3 = _split_w(Wf3, Ws3)
    be1 = jnp.concatenate([bf1, bs1]).reshape(1, 2 * D)
    be2 = jnp.concatenate([bf2, bs2]).reshape(1, 2 * D)
    be3 = jnp.concatenate([bf3, bs3]).reshape(1, 2 * D)

    dtab1, stab1 = _tc_tables(x, Wd1, Wsr1)
    etab1 = _tc_etab(edge_attr, We1, be1)
    agg1 = _sc_edge_pass(dtab1, stab1, etab1, dst, src)
    h1, dtab2, stab2 = _tc_combine_lin(x, agg1, Wl, bl.reshape(1, D),
                                       Wd2, Wsr2)

    etab2 = _tc_etab(edge_attr, We2, be2)
    agg2 = _sc_edge_pass(dtab2, stab2, etab2, dst, src)
    h2, dtab3, stab3 = _tc_combine(h1, agg2, Wd3, Wsr3)

    etab3 = _tc_etab(edge_attr, We3, be3)
    agg3 = _sc_edge_pass(dtab3, stab3, etab3, dst, src)

    sums, cnts = _tc_pool(h2, agg3, batch3d)
    pooled = _tc_head(sums, cnts, Wh1, bh1.reshape(1, D), Wh2,
                      bh2.reshape(1, D), Wh3.reshape(1, D),
                      jnp.broadcast_to(bh3[None, :], (1, D)))
    return pooled[:, :1]


# speedup vs baseline: 1.0374x; 1.0374x over previous
import functools

import jax
import jax.numpy as jnp
from jax import lax
from jax.experimental import pallas as pl
from jax.experimental.pallas import tpu as pltpu
from jax.experimental.pallas import tpu_sc as plsc

N = 10000
E = 640000
D = 128
DE = 16
G = 64

NC = 2
NS = 16
NW = NC * NS
EPW = E // NW
C = 32
NCHUNK = EPW // C
K = 5
NSUP = NCHUNK // K
WTILES = 10
ROWS_PER_WTILE = N // WTILES

_LP6 = -1.72080611e-02
_LP5 = 8.17268084e-02
_LP4 = -1.88782674e-01
_LP3 = 3.14590535e-01
_LP2 = -4.96977911e-01
_LP1 = 9.99792436e-01
_LP0 = 3.50755205e-06


def _sc_edge_body(dtab, stab, etab, dsti, srci, out,
                  didxs, sidxs, dbuf0, dbuf1, sbuf0, sbuf1, ebuf, mbuf,
                  scidx, acc, semd0, semd1, sems0, sems1, seme):
    dbufs = (dbuf0, dbuf1)
    sbufs = (sbuf0, sbuf1)
    semds = (semd0, semd1)
    semss = (sems0, sems1)
    cid = lax.axis_index("c")
    sid = lax.axis_index("s")
    wid = cid * NS + sid

    def _zfill(i, _):
        r = i // 8
        l = (i % 8) * 16
        mbuf[r, pl.ds(l, 16)] = jnp.zeros((16,), jnp.float32)
        return 0
    lax.fori_loop(0, 8 * 8, _zfill, 0)

    @pl.when(sid < WTILES)
    def _zacc_all():
        def _zacc(i, _):
            pltpu.sync_copy(mbuf.at[pl.ds(0, 8)],
                            acc.at[pl.ds(sid * ROWS_PER_WTILE + i * 8, 8)])
            return 0
        lax.fori_loop(0, ROWS_PER_WTILE // 8, _zacc, 0)

    plsc.subcore_barrier()

    crow0 = wid * NCHUNK

    def _issue(sup, cl, slot):
        pltpu.async_copy(dtab.at[didxs.at[cl, 0]], dbufs[slot], semds[slot])
        pltpu.async_copy(stab.at[sidxs.at[cl, 0]], sbufs[slot], semss[slot])

    def _issue_e(sup, cl):
        eb = (crow0 + sup * K + cl) * C
        pltpu.async_copy(etab.at[pl.ds(eb, C)], ebuf, seme)

    def _wait(sup, cl, slot):
        pltpu.make_async_copy(dtab.at[didxs.at[cl, 0]], dbufs[slot],
                              semds[slot]).wait()
        pltpu.make_async_copy(stab.at[sidxs.at[cl, 0]], sbufs[slot],
                              semss[slot]).wait()

    def _wait_e(sup, cl):
        pltpu.make_async_copy(etab.at[pl.ds((crow0 + sup * K + cl) * C, C)],
                              ebuf, seme).wait()

    def _compute(cl, slot):
        db = dbufs[slot]
        sb = sbufs[slot]

        def _edge(e, _):
            for k in range(8):
                lo = k * 16
                f = (db[e, pl.ds(lo, 16)] + sb[e, pl.ds(lo, 16)]
                     + ebuf[e, pl.ds(lo, 16)])
                s = (db[e, pl.ds(128 + lo, 16)]
                     + sb[e, pl.ds(128 + lo, 16)]
                     + ebuf[e, pl.ds(128 + lo, 16)])
                sig = 1.0 / (1.0 + jnp.exp(-f))
                u = jnp.exp(-jnp.abs(s))
                p = _LP6
                p = p * u + _LP5
                p = p * u + _LP4
                p = p * u + _LP3
                p = p * u + _LP2
                p = p * u + _LP1
                p = p * u + _LP0
                sp = jnp.maximum(s, 0.0) + p
                mbuf[e, pl.ds(lo, 16)] = sig * sp
            return 0

        lax.fori_loop(0, C, _edge, 0)
        for w in range(C // 16):
            scidx[pl.ds(16 * w, 16)] = didxs[cl, 0, pl.ds(16 * w, 16)]
        pltpu.sync_copy(mbuf, acc.at[scidx], add=True)

    def _super(sup, _):
        r0 = crow0 + sup * K
        pltpu.sync_copy(dsti.at[pl.ds(r0, K)], didxs)
        pltpu.sync_copy(srci.at[pl.ds(r0, K)], sidxs)
        _issue(sup, 0, 0)
        _issue_e(sup, 0)

        def _pair(p, _):
            cl = 2 * p
            _issue(sup, cl + 1, 1)
            _wait(sup, cl, 0)
            _wait_e(sup, cl)
            _compute(cl, 0)
            _issue_e(sup, cl + 1)
            _issue(sup, cl + 2, 0)
            _wait(sup, cl + 1, 1)
            _wait_e(sup, cl + 1)
            _compute(cl + 1, 1)
            _issue_e(sup, cl + 2)
            return 0

        lax.fori_loop(0, (K - 1) // 2, _pair, 0)
        _wait(sup, K - 1, 0)
        _wait_e(sup, K - 1)
        _compute(K - 1, 0)
        return 0

    lax.fori_loop(0, NSUP, _super, 0)
    plsc.subcore_barrier()

    @pl.when(sid < WTILES)
    def _writeout():
        pltpu.sync_copy(acc.at[pl.ds(sid * ROWS_PER_WTILE, ROWS_PER_WTILE)],
                        out.at[cid, pl.ds(sid * ROWS_PER_WTILE,
                                          ROWS_PER_WTILE)])


_sc_edge_pass = functools.partial(
    pl.kernel,
    out_type=jax.ShapeDtypeStruct((NC, N, D), jnp.float32),
    mesh=plsc.VectorSubcoreMesh(core_axis_name="c", subcore_axis_name="s"),
    scratch_types=[
        pltpu.VMEM((K, 1, C), jnp.int32),
        pltpu.VMEM((K, 1, C), jnp.int32),
        pltpu.VMEM((C, 2 * D), jnp.float32),
        pltpu.VMEM((C, 2 * D), jnp.float32),
        pltpu.VMEM((C, 2 * D), jnp.float32),
        pltpu.VMEM((C, 2 * D), jnp.float32),
        pltpu.VMEM((C, 2 * D), jnp.float32),
        pltpu.VMEM((C, D), jnp.float32),
        pltpu.VMEM((C,), jnp.int32),
        pltpu.VMEM_SHARED((N, D), jnp.float32),
        pltpu.SemaphoreType.DMA,
        pltpu.SemaphoreType.DMA,
        pltpu.SemaphoreType.DMA,
        pltpu.SemaphoreType.DMA,
        pltpu.SemaphoreType.DMA,
    ],
)(_sc_edge_body)


_RB = 1000
_NG = N // _RB
_EB = 2000
_EG = E // _EB


def _tables_body(h_ref, wd_ref, ws_ref, d_ref, s_ref):
    h = h_ref[...]
    d_ref[...] = jnp.dot(h, wd_ref[...], preferred_element_type=jnp.float32)
    s_ref[...] = jnp.dot(h, ws_ref[...], preferred_element_type=jnp.float32)


def _tc_tables(h, Wd, Ws):
    return pl.pallas_call(
        _tables_body,
        grid=(_NG,),
        in_specs=[
            pl.BlockSpec((_RB, D), lambda i: (i, 0)),
            pl.BlockSpec((D, 2 * D), lambda i: (0, 0)),
            pl.BlockSpec((D, 2 * D), lambda i: (0, 0)),
        ],
        out_specs=[pl.BlockSpec((_RB, 2 * D), lambda i: (i, 0))] * 2,
        out_shape=[jax.ShapeDtypeStruct((N, 2 * D), jnp.float32)] * 2,
    )(h, Wd, Ws)


def _etab_body(ea_ref, we_ref, be_ref, o_ref):
    o_ref[...] = (jnp.dot(ea_ref[...], we_ref[...],
                          preferred_element_type=jnp.float32)
                  + be_ref[...])


def _tc_etab(ea, We, be):
    return pl.pallas_call(
        _etab_body,
        grid=(_EG,),
        in_specs=[
            pl.BlockSpec((_EB, DE), lambda i: (i, 0)),
            pl.BlockSpec((DE, 2 * D), lambda i: (0, 0)),
            pl.BlockSpec((1, 2 * D), lambda i: (0, 0)),
        ],
        out_specs=pl.BlockSpec((_EB, 2 * D), lambda i: (i, 0)),
        out_shape=jax.ShapeDtypeStruct((E, 2 * D), jnp.float32),
    )(ea, We, be)


def _comb1_body(x_ref, a_ref, wl_ref, bl_ref, wd_ref, ws_ref,
                h_ref, d_ref, s_ref):
    hr = jnp.maximum(x_ref[...] + a_ref[0] + a_ref[1], 0.0)
    h = jnp.dot(hr, wl_ref[...], preferred_element_type=jnp.float32) + bl_ref[...]
    h_ref[...] = h
    d_ref[...] = jnp.dot(h, wd_ref[...], preferred_element_type=jnp.float32)
    s_ref[...] = jnp.dot(h, ws_ref[...], preferred_element_type=jnp.float32)


def _tc_combine_lin(x, agg, Wl, bl, Wd, Ws):
    return pl.pallas_call(
        _comb1_body,
        grid=(_NG,),
        in_specs=[
            pl.BlockSpec((_RB, D), lambda i: (i, 0)),
            pl.BlockSpec((NC, _RB, D), lambda i: (0, i, 0)),
            pl.BlockSpec((D, D), lambda i: (0, 0)),
            pl.BlockSpec((1, D), lambda i: (0, 0)),
            pl.BlockSpec((D, 2 * D), lambda i: (0, 0)),
            pl.BlockSpec((D, 2 * D), lambda i: (0, 0)),
        ],
        out_specs=[
            pl.BlockSpec((_RB, D), lambda i: (i, 0)),
            pl.BlockSpec((_RB, 2 * D), lambda i: (i, 0)),
            pl.BlockSpec((_RB, 2 * D), lambda i: (i, 0)),
        ],
        out_shape=[
            jax.ShapeDtypeStruct((N, D), jnp.float32),
            jax.ShapeDtypeStruct((N, 2 * D), jnp.float32),
            jax.ShapeDtypeStruct((N, 2 * D), jnp.float32),
        ],
    )(x, agg, Wl, bl, Wd, Ws)


def _comb2_body(x_ref, a_ref, wd_ref, ws_ref, h_ref, d_ref, s_ref):
    h = jnp.maximum(x_ref[...] + a_ref[0] + a_ref[1], 0.0)
    h_ref[...] = h
    d_ref[...] = jnp.dot(h, wd_ref[...], preferred_element_type=jnp.float32)
    s_ref[...] = jnp.dot(h, ws_ref[...], preferred_element_type=jnp.float32)


def _tc_combine(x, agg, Wd, Ws):
    return pl.pallas_call(
        _comb2_body,
        grid=(_NG,),
        in_specs=[
            pl.BlockSpec((_RB, D), lambda i: (i, 0)),
            pl.BlockSpec((NC, _RB, D), lambda i: (0, i, 0)),
            pl.BlockSpec((D, 2 * D), lambda i: (0, 0)),
            pl.BlockSpec((D, 2 * D), lambda i: (0, 0)),
        ],
        out_specs=[
            pl.BlockSpec((_RB, D), lambda i: (i, 0)),
            pl.BlockSpec((_RB, 2 * D), lambda i: (i, 0)),
            pl.BlockSpec((_RB, 2 * D), lambda i: (i, 0)),
        ],
        out_shape=[
            jax.ShapeDtypeStruct((N, D), jnp.float32),
            jax.ShapeDtypeStruct((N, 2 * D), jnp.float32),
            jax.ShapeDtypeStruct((N, 2 * D), jnp.float32),
        ],
    )(x, agg, Wd, Ws)


def _pool_body(x_ref, a_ref, b_ref, sum_ref, cnt_ref):
    i = pl.program_id(0)
    h3 = jnp.maximum(x_ref[...] + a_ref[0] + a_ref[1], 0.0)
    bids = b_ref[0, 0, :]
    oh = (lax.broadcasted_iota(jnp.int32, (G, _RB), 0)
          == bids[None, :]).astype(jnp.float32)
    ps = jnp.dot(oh, h3, preferred_element_type=jnp.float32)
    pc = jnp.broadcast_to(jnp.sum(oh, axis=1)[:, None], (G, D))

    @pl.when(i == 0)
    def _():
        sum_ref[...] = ps
        cnt_ref[...] = pc

    @pl.when(i > 0)
    def _():
        sum_ref[...] += ps
        cnt_ref[...] += pc


def _tc_pool(h2, agg, batch3d):
    return pl.pallas_call(
        _pool_body,
        grid=(_NG,),
        in_specs=[
            pl.BlockSpec((_RB, D), lambda i: (i, 0)),
            pl.BlockSpec((NC, _RB, D), lambda i: (0, i, 0)),
            pl.BlockSpec((1, 1, _RB), lambda i: (i, 0, 0)),
        ],
        out_specs=[pl.BlockSpec((G, D), lambda i: (0, 0))] * 2,
        out_shape=[jax.ShapeDtypeStruct((G, D), jnp.float32)] * 2,
    )(h2, agg, batch3d)


def _head_body(s_ref, c_ref, w1_ref, b1_ref, w2_ref, b2_ref, w3_ref, b3_ref,
               o_ref):
    g = s_ref[...] / jnp.maximum(c_ref[...], 1.0)
    g = jnp.maximum(jnp.dot(g, w1_ref[...],
                            preferred_element_type=jnp.float32) + b1_ref[...],
                    0.0)
    g = jnp.maximum(jnp.dot(g, w2_ref[...],
                            preferred_element_type=jnp.float32) + b2_ref[...],
                    0.0)
    o = jnp.sum(g * w3_ref[...], axis=1, keepdims=True) + b3_ref[0, 0]
    o_ref[...] = jnp.broadcast_to(o, (G, D))


def _tc_head(sums, cnts, Wh1, bh1, Wh2, bh2, w3row, bh3):
    return pl.pallas_call(
        _head_body,
        in_specs=[
            pl.BlockSpec((G, D), lambda: (0, 0)),
            pl.BlockSpec((G, D), lambda: (0, 0)),
            pl.BlockSpec((D, D), lambda: (0, 0)),
            pl.BlockSpec((1, D), lambda: (0, 0)),
            pl.BlockSpec((D, D), lambda: (0, 0)),
            pl.BlockSpec((1, D), lambda: (0, 0)),
            pl.BlockSpec((1, D), lambda: (0, 0)),
            pl.BlockSpec((1, D), lambda: (0, 0)),
        ],
        out_specs=pl.BlockSpec((G, D), lambda: (0, 0)),
        out_shape=jax.ShapeDtypeStruct((G, D), jnp.float32),
    )(sums, cnts, Wh1, bh1, Wh2, bh2, w3row, bh3)


def _split_w(Wf, Ws):
    Wd = jnp.concatenate([Wf[:D], Ws[:D]], axis=1)
    Wsrc = jnp.concatenate([Wf[D:2 * D], Ws[D:2 * D]], axis=1)
    We = jnp.concatenate([Wf[2 * D:], Ws[2 * D:]], axis=1)
    return Wd, Wsrc, We


def kernel(x, edge_index, edge_attr, batch,
           Wf1, bf1, Ws1, bs1, Wl, bl,
           Wf2, bf2, Ws2, bs2, Wf3, bf3, Ws3, bs3,
           Wh1, bh1, Wh2, bh2, Wh3, bh3):
    src = edge_index[0].astype(jnp.int32).reshape(E // C, 1, C)
    dst = edge_index[1].astype(jnp.int32).reshape(E // C, 1, C)
    batch3d = batch.astype(jnp.int32).reshape(_NG, 1, _RB)

    Wd1, Wsr1, We1 = _split_w(Wf1, Ws1)
    Wd2, Wsr2, We2 = _split_w(Wf2, Ws2)
    Wd3, Wsr3, We3 = _split_w(Wf3, Ws3)
    be1 = jnp.concatenate([bf1, bs1]).reshape(1, 2 * D)
    be2 = jnp.concatenate([bf2, bs2]).reshape(1, 2 * D)
    be3 = jnp.concatenate([bf3, bs3]).reshape(1, 2 * D)

    dtab1, stab1 = _tc_tables(x, Wd1, Wsr1)
    etab1 = _tc_etab(edge_attr, We1, be1)
    agg1 = _sc_edge_pass(dtab1, stab1, etab1, dst, src)
    h1, dtab2, stab2 = _tc_combine_lin(x, agg1, Wl, bl.reshape(1, D),
                                       Wd2, Wsr2)

    etab2 = _tc_etab(edge_attr, We2, be2)
    agg2 = _sc_edge_pass(dtab2, stab2, etab2, dst, src)
    h2, dtab3, stab3 = _tc_combine(h1, agg2, Wd3, Wsr3)

    etab3 = _tc_etab(edge_attr, We3, be3)
    agg3 = _sc_edge_pass(dtab3, stab3, etab3, dst, src)

    sums, cnts = _tc_pool(h2, agg3, batch3d)
    pooled = _tc_head(sums, cnts, Wh1, bh1.reshape(1, D), Wh2,
                      bh2.reshape(1, D), Wh3.reshape(1, D),
                      jnp.broadcast_to(bh3[None, :], (1, D)))
    return pooled[:, :1]


# speedup vs baseline: 5.9599x; 5.7451x over previous
import functools

import jax
import jax.numpy as jnp
from jax import lax
from jax.experimental import pallas as pl
from jax.experimental.pallas import tpu as pltpu
from jax.experimental.pallas import tpu_sc as plsc

N = 10000
E = 640000
D = 128
DE = 16
G = 64

NC = 2
NS = 16
NW = NC * NS
EPW = E // NW
CG = 200
NCHUNK = EPW // CG
K2 = 20
NBLK = NCHUNK // K2
NPAIR = K2 // 2
WTILES = 10
RPT = N // WTILES

CG2 = 80
NCHUNK2 = EPW // CG2
K2S = 10
NBLK2 = NCHUNK2 // K2S
NPAIR2 = K2S // 2


def _sc_gather_body(h, dsti, srci, gd, gs,
                    didxs, sidxs, db0, db1, sb0, sb1,
                    semd0, semd1, sems0, sems1):
    cid = lax.axis_index("c")
    sid = lax.axis_index("s")
    wid = cid * NS + sid
    crow0 = wid * NCHUNK
    dbufs = (db0, db1)
    sbufs = (sb0, sb1)
    semds = (semd0, semd1)
    semss = (sems0, sems1)

    def _issue(cl, slot):
        pltpu.async_copy(h.at[didxs.at[cl, 0]], dbufs[slot], semds[slot])
        pltpu.async_copy(h.at[sidxs.at[cl, 0]], sbufs[slot], semss[slot])

    def _wait(cl, slot):
        pltpu.make_async_copy(h.at[didxs.at[cl, 0]], dbufs[slot],
                              semds[slot]).wait()
        pltpu.make_async_copy(h.at[sidxs.at[cl, 0]], sbufs[slot],
                              semss[slot]).wait()

    def _writeout(b, cl, slot):
        eb = (crow0 + b * K2 + cl) * CG
        pltpu.sync_copy(dbufs[slot], gd.at[pl.ds(eb, CG)])
        pltpu.sync_copy(sbufs[slot], gs.at[pl.ds(eb, CG)])

    def _block(b, _):
        r0 = crow0 + b * K2
        pltpu.sync_copy(dsti.at[pl.ds(r0, K2)], didxs)
        pltpu.sync_copy(srci.at[pl.ds(r0, K2)], sidxs)
        _issue(0, 0)

        def _pair(p, _):
            cl = 2 * p
            _issue(cl + 1, 1)
            _wait(cl, 0)
            _writeout(b, cl, 0)

            @pl.when(p < NPAIR - 1)
            def _():
                _issue(cl + 2, 0)

            _wait(cl + 1, 1)
            _writeout(b, cl + 1, 1)
            return 0

        lax.fori_loop(0, NPAIR, _pair, 0)
        return 0

    lax.fori_loop(0, NBLK, _block, 0)


_sc_gather = functools.partial(
    pl.kernel,
    out_type=[jax.ShapeDtypeStruct((E, D), jnp.float32)] * 2,
    mesh=plsc.VectorSubcoreMesh(core_axis_name="c", subcore_axis_name="s"),
    scratch_types=[
        pltpu.VMEM((K2, 1, CG), jnp.int32),
        pltpu.VMEM((K2, 1, CG), jnp.int32),
        pltpu.VMEM((CG, D), jnp.float32),
        pltpu.VMEM((CG, D), jnp.float32),
        pltpu.VMEM((CG, D), jnp.float32),
        pltpu.VMEM((CG, D), jnp.float32),
        pltpu.SemaphoreType.DMA,
        pltpu.SemaphoreType.DMA,
        pltpu.SemaphoreType.DMA,
        pltpu.SemaphoreType.DMA,
    ],
)(_sc_gather_body)


def _sc_scatter_body(m, dsti, out,
                     didxs, mb0, mb1, zb, acc, semm0, semm1):
    cid = lax.axis_index("c")
    sid = lax.axis_index("s")
    wid = cid * NS + sid
    crow0 = wid * NCHUNK2
    mbufs = (mb0, mb1)
    semms = (semm0, semm1)

    def _zfill(i, _):
        r = i // 8
        l = (i % 8) * 16
        zb[r, pl.ds(l, 16)] = jnp.zeros((16,), jnp.float32)
        return 0
    lax.fori_loop(0, 50 * 8, _zfill, 0)

    @pl.when(sid < WTILES)
    def _zacc_all():
        def _zacc(i, _):
            pltpu.sync_copy(zb, acc.at[pl.ds(sid * RPT + i * 50, 50)])
            return 0
        lax.fori_loop(0, RPT // 50, _zacc, 0)

    plsc.subcore_barrier()

    def _issue(cl, slot):
        eb = (crow0 + cl) * CG2
        pltpu.async_copy(m.at[pl.ds(eb, CG2)], mbufs[slot], semms[slot])

    def _wait(cl, slot):
        eb = (crow0 + cl) * CG2
        pltpu.make_async_copy(m.at[pl.ds(eb, CG2)], mbufs[slot],
                              semms[slot]).wait()

    def _scat(cl, slot):
        pltpu.sync_copy(mbufs[slot], acc.at[didxs.at[cl, 0]], add=True)

    def _block(b, _):
        r0 = crow0 + b * K2S
        pltpu.sync_copy(dsti.at[pl.ds(r0, K2S)], didxs)
        _issue(b * K2S, 0)

        def _pair(p, _):
            cl = 2 * p
            _issue(b * K2S + cl + 1, 1)
            _wait(b * K2S + cl, 0)
            _scat(cl, 0)

            @pl.when(p < NPAIR2 - 1)
            def _():
                _issue(b * K2S + cl + 2, 0)

            _wait(b * K2S + cl + 1, 1)
            _scat(cl + 1, 1)
            return 0

        lax.fori_loop(0, NPAIR2, _pair, 0)
        return 0

    lax.fori_loop(0, NBLK2, _block, 0)
    plsc.subcore_barrier()

    @pl.when(sid < WTILES)
    def _writeout():
        pltpu.sync_copy(acc.at[pl.ds(sid * RPT, RPT)],
                        out.at[cid, pl.ds(sid * RPT, RPT)])


_sc_scatter = functools.partial(
    pl.kernel,
    out_type=jax.ShapeDtypeStruct((NC, N, D), jnp.float32),
    mesh=plsc.VectorSubcoreMesh(core_axis_name="c", subcore_axis_name="s"),
    scratch_types=[
        pltpu.VMEM((K2S, 1, CG2), jnp.int32),
        pltpu.VMEM((CG2, D), jnp.float32),
        pltpu.VMEM((CG2, D), jnp.float32),
        pltpu.VMEM((50, D), jnp.float32),
        pltpu.VMEM_SHARED((N, D), jnp.float32),
        pltpu.SemaphoreType.DMA,
        pltpu.SemaphoreType.DMA,
    ],
)(_sc_scatter_body)


_RB = 1000
_NG = N // _RB
_EB = 4000
_EG = E // _EB


def _edge_body(gd_ref, gs_ref, ea_ref, wd_ref, ws_ref, we_ref, b_ref, m_ref):
    fs = (jnp.dot(gd_ref[...], wd_ref[...],
                  preferred_element_type=jnp.float32)
          + jnp.dot(gs_ref[...], ws_ref[...],
                    preferred_element_type=jnp.float32)
          + jnp.dot(ea_ref[...], we_ref[...],
                    preferred_element_type=jnp.float32)
          + b_ref[...])
    f = fs[:, :D]
    s = fs[:, D:]
    sig = 1.0 / (1.0 + jnp.exp(-f))
    sp = jnp.maximum(s, 0.0) + jnp.log1p(jnp.exp(-jnp.abs(s)))
    m_ref[...] = sig * sp


def _tc_edge(gd, gs, ea, Wd, Ws, We, b):
    return pl.pallas_call(
        _edge_body,
        grid=(_EG,),
        in_specs=[
            pl.BlockSpec((_EB, D), lambda i: (i, 0)),
            pl.BlockSpec((_EB, D), lambda i: (i, 0)),
            pl.BlockSpec((_EB, DE), lambda i: (i, 0)),
            pl.BlockSpec((D, 2 * D), lambda i: (0, 0)),
            pl.BlockSpec((D, 2 * D), lambda i: (0, 0)),
            pl.BlockSpec((DE, 2 * D), lambda i: (0, 0)),
            pl.BlockSpec((1, 2 * D), lambda i: (0, 0)),
        ],
        out_specs=pl.BlockSpec((_EB, D), lambda i: (i, 0)),
        out_shape=jax.ShapeDtypeStruct((E, D), jnp.float32),
    )(gd, gs, ea, Wd, Ws, We, b)


def _comb1_body(x_ref, a_ref, wl_ref, bl_ref, h_ref):
    hr = jnp.maximum(x_ref[...] + a_ref[0] + a_ref[1], 0.0)
    h_ref[...] = (jnp.dot(hr, wl_ref[...], preferred_element_type=jnp.float32)
                  + bl_ref[...])


def _tc_combine_lin(x, agg, Wl, bl):
    return pl.pallas_call(
        _comb1_body,
        grid=(_NG,),
        in_specs=[
            pl.BlockSpec((_RB, D), lambda i: (i, 0)),
            pl.BlockSpec((NC, _RB, D), lambda i: (0, i, 0)),
            pl.BlockSpec((D, D), lambda i: (0, 0)),
            pl.BlockSpec((1, D), lambda i: (0, 0)),
        ],
        out_specs=pl.BlockSpec((_RB, D), lambda i: (i, 0)),
        out_shape=jax.ShapeDtypeStruct((N, D), jnp.float32),
    )(x, agg, Wl, bl)


def _comb2_body(x_ref, a_ref, h_ref):
    h_ref[...] = jnp.maximum(x_ref[...] + a_ref[0] + a_ref[1], 0.0)


def _tc_combine(x, agg):
    return pl.pallas_call(
        _comb2_body,
        grid=(_NG,),
        in_specs=[
            pl.BlockSpec((_RB, D), lambda i: (i, 0)),
            pl.BlockSpec((NC, _RB, D), lambda i: (0, i, 0)),
        ],
        out_specs=pl.BlockSpec((_RB, D), lambda i: (i, 0)),
        out_shape=jax.ShapeDtypeStruct((N, D), jnp.float32),
    )(x, agg)


def _pool_body(x_ref, a_ref, b_ref, sum_ref, cnt_ref):
    i = pl.program_id(0)
    h3 = jnp.maximum(x_ref[...] + a_ref[0] + a_ref[1], 0.0)
    bids = b_ref[0, 0, :]
    oh = (lax.broadcasted_iota(jnp.int32, (G, _RB), 0)
          == bids[None, :]).astype(jnp.float32)
    ps = jnp.dot(oh, h3, preferred_element_type=jnp.float32)
    pc = jnp.broadcast_to(jnp.sum(oh, axis=1)[:, None], (G, D))

    @pl.when(i == 0)
    def _():
        sum_ref[...] = ps
        cnt_ref[...] = pc

    @pl.when(i > 0)
    def _():
        sum_ref[...] += ps
        cnt_ref[...] += pc


def _tc_pool(h2, agg, batch3d):
    return pl.pallas_call(
        _pool_body,
        grid=(_NG,),
        in_specs=[
            pl.BlockSpec((_RB, D), lambda i: (i, 0)),
            pl.BlockSpec((NC, _RB, D), lambda i: (0, i, 0)),
            pl.BlockSpec((1, 1, _RB), lambda i: (i, 0, 0)),
        ],
        out_specs=[pl.BlockSpec((G, D), lambda i: (0, 0))] * 2,
        out_shape=[jax.ShapeDtypeStruct((G, D), jnp.float32)] * 2,
    )(h2, agg, batch3d)


def _head_body(s_ref, c_ref, w1_ref, b1_ref, w2_ref, b2_ref, w3_ref, b3_ref,
               o_ref):
    g = s_ref[...] / jnp.maximum(c_ref[...], 1.0)
    g = jnp.maximum(jnp.dot(g, w1_ref[...],
                            preferred_element_type=jnp.float32) + b1_ref[...],
                    0.0)
    g = jnp.maximum(jnp.dot(g, w2_ref[...],
                            preferred_element_type=jnp.float32) + b2_ref[...],
                    0.0)
    o = jnp.sum(g * w3_ref[...], axis=1, keepdims=True) + b3_ref[0, 0]
    o_ref[...] = jnp.broadcast_to(o, (G, D))


def _tc_head(sums, cnts, Wh1, bh1, Wh2, bh2, w3row, bh3):
    return pl.pallas_call(
        _head_body,
        in_specs=[
            pl.BlockSpec((G, D), lambda: (0, 0)),
            pl.BlockSpec((G, D), lambda: (0, 0)),
            pl.BlockSpec((D, D), lambda: (0, 0)),
            pl.BlockSpec((1, D), lambda: (0, 0)),
            pl.BlockSpec((D, D), lambda: (0, 0)),
            pl.BlockSpec((1, D), lambda: (0, 0)),
            pl.BlockSpec((1, D), lambda: (0, 0)),
            pl.BlockSpec((1, D), lambda: (0, 0)),
        ],
        out_specs=pl.BlockSpec((G, D), lambda: (0, 0)),
        out_shape=jax.ShapeDtypeStruct((G, D), jnp.float32),
    )(sums, cnts, Wh1, bh1, Wh2, bh2, w3row, bh3)


def _split_w(Wf, Ws):
    Wd = jnp.concatenate([Wf[:D], Ws[:D]], axis=1)
    Wsrc = jnp.concatenate([Wf[D:2 * D], Ws[D:2 * D]], axis=1)
    We = jnp.concatenate([Wf[2 * D:], Ws[2 * D:]], axis=1)
    return Wd, Wsrc, We


def kernel(x, edge_index, edge_attr, batch,
           Wf1, bf1, Ws1, bs1, Wl, bl,
           Wf2, bf2, Ws2, bs2, Wf3, bf3, Ws3, bs3,
           Wh1, bh1, Wh2, bh2, Wh3, bh3):
    src = edge_index[0].astype(jnp.int32).reshape(E // CG, 1, CG)
    dst = edge_index[1].astype(jnp.int32).reshape(E // CG, 1, CG)
    dst2 = edge_index[1].astype(jnp.int32).reshape(E // CG2, 1, CG2)
    batch3d = batch.astype(jnp.int32).reshape(_NG, 1, _RB)

    Wd1, Wsr1, We1 = _split_w(Wf1, Ws1)
    Wd2, Wsr2, We2 = _split_w(Wf2, Ws2)
    Wd3, Wsr3, We3 = _split_w(Wf3, Ws3)
    b1 = jnp.concatenate([bf1, bs1]).reshape(1, 2 * D)
    b2 = jnp.concatenate([bf2, bs2]).reshape(1, 2 * D)
    b3 = jnp.concatenate([bf3, bs3]).reshape(1, 2 * D)

    gd1, gs1 = _sc_gather(x, dst, src)
    m1 = _tc_edge(gd1, gs1, edge_attr, Wd1, Wsr1, We1, b1)
    agg1 = _sc_scatter(m1, dst2)
    h2in = _tc_combine_lin(x, agg1, Wl, bl.reshape(1, D))

    gd2, gs2 = _sc_gather(h2in, dst, src)
    m2 = _tc_edge(gd2, gs2, edge_attr, Wd2, Wsr2, We2, b2)
    agg2 = _sc_scatter(m2, dst2)
    h2 = _tc_combine(h2in, agg2)

    gd3, gs3 = _sc_gather(h2, dst, src)
    m3 = _tc_edge(gd3, gs3, edge_attr, Wd3, Wsr3, We3, b3)
    agg3 = _sc_scatter(m3, dst2)

    sums, cnts = _tc_pool(h2, agg3, batch3d)
    pooled = _tc_head(sums, cnts, Wh1, bh1.reshape(1, D), Wh2,
                      bh2.reshape(1, D), Wh3.reshape(1, D),
                      jnp.broadcast_to(bh3[None, :], (1, D)))
    return pooled[:, :1]


# speedup vs baseline: 6.2114x; 1.0422x over previous
import functools

import jax
import jax.numpy as jnp
from jax import lax
from jax.experimental import pallas as pl
from jax.experimental.pallas import tpu as pltpu
from jax.experimental.pallas import tpu_sc as plsc

N = 10000
E = 640000
EH = E // 2
D = 128
DE = 16
G = 64

NC = 2
NS = 16
NW = NC * NS
CG = 200
KG = 10
CS = 40
KS = 10
WTILES = 10
RPT = N // WTILES


def _gather_body(nchunk, nblk, npair,
                 h, dsti, srci, gd, gs,
                 didxs, sidxs, db0, db1, sb0, sb1,
                 semd0, semd1, sems0, sems1):
    cid = lax.axis_index("c")
    sid = lax.axis_index("s")
    wid = cid * NS + sid
    crow0 = wid * nchunk
    dbufs = (db0, db1)
    sbufs = (sb0, sb1)
    semds = (semd0, semd1)
    semss = (sems0, sems1)

    def _issue(cl, slot):
        pltpu.async_copy(h.at[didxs.at[cl, 0]], dbufs[slot], semds[slot])
        pltpu.async_copy(h.at[sidxs.at[cl, 0]], sbufs[slot], semss[slot])

    def _wait(cl, slot):
        pltpu.make_async_copy(h.at[didxs.at[cl, 0]], dbufs[slot],
                              semds[slot]).wait()
        pltpu.make_async_copy(h.at[sidxs.at[cl, 0]], sbufs[slot],
                              semss[slot]).wait()

    def _writeout(b, cl, slot):
        eb = (crow0 + b * KG + cl) * CG
        pltpu.sync_copy(dbufs[slot], gd.at[pl.ds(eb, CG)])
        pltpu.sync_copy(sbufs[slot], gs.at[pl.ds(eb, CG)])

    def _block(b, _):
        r0 = crow0 + b * KG
        pltpu.sync_copy(dsti.at[pl.ds(r0, KG)], didxs)
        pltpu.sync_copy(srci.at[pl.ds(r0, KG)], sidxs)
        _issue(0, 0)

        def _pair(p, _):
            cl = 2 * p
            _issue(cl + 1, 1)
            _wait(cl, 0)
            _writeout(b, cl, 0)

            @pl.when(p < npair - 1)
            def _():
                _issue(cl + 2, 0)

            _wait(cl + 1, 1)
            _writeout(b, cl + 1, 1)
            return 0

        lax.fori_loop(0, npair, _pair, 0)
        return 0

    lax.fori_loop(0, nblk, _block, 0)


def _make_gather(ne):
    nchunk = ne // NW // CG
    nblk = nchunk // KG
    return functools.partial(
        pl.kernel,
        out_type=[jax.ShapeDtypeStruct((ne, D), jnp.float32)] * 2,
        mesh=plsc.VectorSubcoreMesh(core_axis_name="c", subcore_axis_name="s"),
        scratch_types=[
            pltpu.VMEM((KG, 1, CG), jnp.int32),
            pltpu.VMEM((KG, 1, CG), jnp.int32),
            pltpu.VMEM((CG, D), jnp.float32),
            pltpu.VMEM((CG, D), jnp.float32),
            pltpu.VMEM((CG, D), jnp.float32),
            pltpu.VMEM((CG, D), jnp.float32),
            pltpu.SemaphoreType.DMA,
            pltpu.SemaphoreType.DMA,
            pltpu.SemaphoreType.DMA,
            pltpu.SemaphoreType.DMA,
        ],
    )(functools.partial(_gather_body, nchunk, nblk, KG // 2))


_sc_gather_h = _make_gather(EH)


def _scatter_body(nchunk, nblk, npair,
                  m, dsti, out, didxs, mb0, mb1, zb, acc, semm0, semm1):
    cid = lax.axis_index("c")
    sid = lax.axis_index("s")
    wid = cid * NS + sid
    crow0 = wid * nchunk
    mbufs = (mb0, mb1)
    semms = (semm0, semm1)

    def _zfill(i, _):
        r = i // 8
        l = (i % 8) * 16
        zb[r, pl.ds(l, 16)] = jnp.zeros((16,), jnp.float32)
        return 0
    lax.fori_loop(0, 50 * 8, _zfill, 0)

    @pl.when(sid < WTILES)
    def _zacc_all():
        def _zacc(i, _):
            pltpu.sync_copy(zb, acc.at[pl.ds(sid * RPT + i * 50, 50)])
            return 0
        lax.fori_loop(0, RPT // 50, _zacc, 0)

    plsc.subcore_barrier()

    def _issue(cl, slot):
        eb = (crow0 + cl) * CS
        pltpu.async_copy(m.at[pl.ds(eb, CS)], mbufs[slot], semms[slot])

    def _wait(cl, slot):
        eb = (crow0 + cl) * CS
        pltpu.make_async_copy(m.at[pl.ds(eb, CS)], mbufs[slot],
                              semms[slot]).wait()

    def _scat(cl, slot):
        pltpu.sync_copy(mbufs[slot], acc.at[didxs.at[cl, 0]], add=True)

    def _block(b, _):
        r0 = crow0 + b * KS
        pltpu.sync_copy(dsti.at[pl.ds(r0, KS)], didxs)
        _issue(b * KS, 0)

        def _pair(p, _):
            cl = 2 * p
            _issue(b * KS + cl + 1, 1)
            _wait(b * KS + cl, 0)
            _scat(cl, 0)

            @pl.when(p < npair - 1)
            def _():
                _issue(b * KS + cl + 2, 0)

            _wait(b * KS + cl + 1, 1)
            _scat(cl + 1, 1)
            return 0

        lax.fori_loop(0, npair, _pair, 0)
        return 0

    lax.fori_loop(0, nblk, _block, 0)
    plsc.subcore_barrier()

    @pl.when(sid < WTILES)
    def _writeout():
        pltpu.sync_copy(acc.at[pl.ds(sid * RPT, RPT)],
                        out.at[cid, pl.ds(sid * RPT, RPT)])


def _make_scatter(ne):
    nchunk = ne // NW // CS
    nblk = nchunk // KS
    return functools.partial(
        pl.kernel,
        out_type=jax.ShapeDtypeStruct((NC, N, D), jnp.float32),
        mesh=plsc.VectorSubcoreMesh(core_axis_name="c", subcore_axis_name="s"),
        scratch_types=[
            pltpu.VMEM((KS, 1, CS), jnp.int32),
            pltpu.VMEM((CS, D), jnp.float32),
            pltpu.VMEM((CS, D), jnp.float32),
            pltpu.VMEM((50, D), jnp.float32),
            pltpu.VMEM_SHARED((N, D), jnp.float32),
            pltpu.SemaphoreType.DMA,
            pltpu.SemaphoreType.DMA,
        ],
    )(functools.partial(_scatter_body, nchunk, nblk, KS // 2))


_sc_scatter_h = _make_scatter(EH)


_RB = 1000
_NG = N // _RB
_EB = 4000
_EGH = EH // _EB


def _edge_body(gd_ref, gs_ref, ea_ref, wd_ref, ws_ref, we_ref, b_ref, m_ref):
    fs = (jnp.dot(gd_ref[...], wd_ref[...],
                  preferred_element_type=jnp.float32)
          + jnp.dot(gs_ref[...], ws_ref[...],
                    preferred_element_type=jnp.float32)
          + jnp.dot(ea_ref[...], we_ref[...],
                    preferred_element_type=jnp.float32)
          + b_ref[...])
    f = fs[:, :D]
    s = fs[:, D:]
    sig = 1.0 / (1.0 + jnp.exp(-f))
    sp = jnp.maximum(s, 0.0) + jnp.log1p(jnp.exp(-jnp.abs(s)))
    m_ref[...] = sig * sp


def _tc_edge_h(gd, gs, ea, Wd, Ws, We, b):
    return pl.pallas_call(
        _edge_body,
        grid=(_EGH,),
        in_specs=[
            pl.BlockSpec((_EB, D), lambda i: (i, 0)),
            pl.BlockSpec((_EB, D), lambda i: (i, 0)),
            pl.BlockSpec((_EB, DE), lambda i: (i, 0)),
            pl.BlockSpec((D, 2 * D), lambda i: (0, 0)),
            pl.BlockSpec((D, 2 * D), lambda i: (0, 0)),
            pl.BlockSpec((DE, 2 * D), lambda i: (0, 0)),
            pl.BlockSpec((1, 2 * D), lambda i: (0, 0)),
        ],
        out_specs=pl.BlockSpec((_EB, D), lambda i: (i, 0)),
        out_shape=jax.ShapeDtypeStruct((EH, D), jnp.float32),
    )(gd, gs, ea, Wd, Ws, We, b)


def _comb1_body(x_ref, a_ref, a2_ref, wl_ref, bl_ref, h_ref):
    hr = jnp.maximum(x_ref[...] + a_ref[0] + a_ref[1]
                     + a2_ref[0] + a2_ref[1], 0.0)
    h_ref[...] = (jnp.dot(hr, wl_ref[...], preferred_element_type=jnp.float32)
                  + bl_ref[...])


def _tc_combine_lin(x, agg, agg2, Wl, bl):
    return pl.pallas_call(
        _comb1_body,
        grid=(_NG,),
        in_specs=[
            pl.BlockSpec((_RB, D), lambda i: (i, 0)),
            pl.BlockSpec((NC, _RB, D), lambda i: (0, i, 0)),
            pl.BlockSpec((NC, _RB, D), lambda i: (0, i, 0)),
            pl.BlockSpec((D, D), lambda i: (0, 0)),
            pl.BlockSpec((1, D), lambda i: (0, 0)),
        ],
        out_specs=pl.BlockSpec((_RB, D), lambda i: (i, 0)),
        out_shape=jax.ShapeDtypeStruct((N, D), jnp.float32),
    )(x, agg, agg2, Wl, bl)


def _comb2_body(x_ref, a_ref, a2_ref, h_ref):
    h_ref[...] = jnp.maximum(x_ref[...] + a_ref[0] + a_ref[1]
                             + a2_ref[0] + a2_ref[1], 0.0)


def _tc_combine(x, agg, agg2):
    return pl.pallas_call(
        _comb2_body,
        grid=(_NG,),
        in_specs=[
            pl.BlockSpec((_RB, D), lambda i: (i, 0)),
            pl.BlockSpec((NC, _RB, D), lambda i: (0, i, 0)),
            pl.BlockSpec((NC, _RB, D), lambda i: (0, i, 0)),
        ],
        out_specs=pl.BlockSpec((_RB, D), lambda i: (i, 0)),
        out_shape=jax.ShapeDtypeStruct((N, D), jnp.float32),
    )(x, agg, agg2)


def _pool_body(x_ref, a_ref, a2_ref, b_ref, sum_ref, cnt_ref):
    i = pl.program_id(0)
    h3 = jnp.maximum(x_ref[...] + a_ref[0] + a_ref[1]
                     + a2_ref[0] + a2_ref[1], 0.0)
    bids = b_ref[0, 0, :]
    oh = (lax.broadcasted_iota(jnp.int32, (G, _RB), 0)
          == bids[None, :]).astype(jnp.float32)
    ps = jnp.dot(oh, h3, preferred_element_type=jnp.float32)
    pc = jnp.broadcast_to(jnp.sum(oh, axis=1)[:, None], (G, D))

    @pl.when(i == 0)
    def _():
        sum_ref[...] = ps
        cnt_ref[...] = pc

    @pl.when(i > 0)
    def _():
        sum_ref[...] += ps
        cnt_ref[...] += pc


def _tc_pool(h2, agg, agg2, batch3d):
    return pl.pallas_call(
        _pool_body,
        grid=(_NG,),
        in_specs=[
            pl.BlockSpec((_RB, D), lambda i: (i, 0)),
            pl.BlockSpec((NC, _RB, D), lambda i: (0, i, 0)),
            pl.BlockSpec((NC, _RB, D), lambda i: (0, i, 0)),
            pl.BlockSpec((1, 1, _RB), lambda i: (i, 0, 0)),
        ],
        out_specs=[pl.BlockSpec((G, D), lambda i: (0, 0))] * 2,
        out_shape=[jax.ShapeDtypeStruct((G, D), jnp.float32)] * 2,
    )(h2, agg, agg2, batch3d)


def _head_body(s_ref, c_ref, w1_ref, b1_ref, w2_ref, b2_ref, w3_ref, b3_ref,
               o_ref):
    g = s_ref[...] / jnp.maximum(c_ref[...], 1.0)
    g = jnp.maximum(jnp.dot(g, w1_ref[...],
                            preferred_element_type=jnp.float32) + b1_ref[...],
                    0.0)
    g = jnp.maximum(jnp.dot(g, w2_ref[...],
                            preferred_element_type=jnp.float32) + b2_ref[...],
                    0.0)
    o = jnp.sum(g * w3_ref[...], axis=1, keepdims=True) + b3_ref[0, 0]
    o_ref[...] = jnp.broadcast_to(o, (G, D))


def _tc_head(sums, cnts, Wh1, bh1, Wh2, bh2, w3row, bh3):
    return pl.pallas_call(
        _head_body,
        in_specs=[
            pl.BlockSpec((G, D), lambda: (0, 0)),
            pl.BlockSpec((G, D), lambda: (0, 0)),
            pl.BlockSpec((D, D), lambda: (0, 0)),
            pl.BlockSpec((1, D), lambda: (0, 0)),
            pl.BlockSpec((D, D), lambda: (0, 0)),
            pl.BlockSpec((1, D), lambda: (0, 0)),
            pl.BlockSpec((1, D), lambda: (0, 0)),
            pl.BlockSpec((1, D), lambda: (0, 0)),
        ],
        out_specs=pl.BlockSpec((G, D), lambda: (0, 0)),
        out_shape=jax.ShapeDtypeStruct((G, D), jnp.float32),
    )(sums, cnts, Wh1, bh1, Wh2, bh2, w3row, bh3)


def _split_w(Wf, Ws):
    Wd = jnp.concatenate([Wf[:D], Ws[:D]], axis=1)
    Wsrc = jnp.concatenate([Wf[D:2 * D], Ws[D:2 * D]], axis=1)
    We = jnp.concatenate([Wf[2 * D:], Ws[2 * D:]], axis=1)
    return Wd, Wsrc, We


def _layer(h, idx, ea_h, Wd, Wsr, We, b):
    (dstg, srcg, dsts) = idx
    aggs = []
    for half in (0, 1):
        gd, gs = _sc_gather_h(h, dstg[half], srcg[half])
        m = _tc_edge_h(gd, gs, ea_h[half], Wd, Wsr, We, b)
        aggs.append(_sc_scatter_h(m, dsts[half]))
    return aggs


def kernel(x, edge_index, edge_attr, batch,
           Wf1, bf1, Ws1, bs1, Wl, bl,
           Wf2, bf2, Ws2, bs2, Wf3, bf3, Ws3, bs3,
           Wh1, bh1, Wh2, bh2, Wh3, bh3):
    srcg = edge_index[0].astype(jnp.int32).reshape(E // CG, 1, CG)
    dstg = edge_index[1].astype(jnp.int32).reshape(E // CG, 1, CG)
    dsts = edge_index[1].astype(jnp.int32).reshape(E // CS, 1, CS)
    hg = EH // CG
    hs = EH // CS
    idx = ((dstg[:hg], dstg[hg:]), (srcg[:hg], srcg[hg:]),
           (dsts[:hs], dsts[hs:]))
    ea_h = (edge_attr[:EH], edge_attr[EH:])
    batch3d = batch.astype(jnp.int32).reshape(_NG, 1, _RB)

    Wd1, Wsr1, We1 = _split_w(Wf1, Ws1)
    Wd2, Wsr2, We2 = _split_w(Wf2, Ws2)
    Wd3, Wsr3, We3 = _split_w(Wf3, Ws3)
    b1 = jnp.concatenate([bf1, bs1]).reshape(1, 2 * D)
    b2 = jnp.concatenate([bf2, bs2]).reshape(1, 2 * D)
    b3 = jnp.concatenate([bf3, bs3]).reshape(1, 2 * D)

    agg1a, agg1b = _layer(x, idx, ea_h, Wd1, Wsr1, We1, b1)
    h2in = _tc_combine_lin(x, agg1a, agg1b, Wl, bl.reshape(1, D))

    agg2a, agg2b = _layer(h2in, idx, ea_h, Wd2, Wsr2, We2, b2)
    h2 = _tc_combine(h2in, agg2a, agg2b)

    agg3a, agg3b = _layer(h2, idx, ea_h, Wd3, Wsr3, We3, b3)

    sums, cnts = _tc_pool(h2, agg3a, agg3b, batch3d)
    pooled = _tc_head(sums, cnts, Wh1, bh1.reshape(1, D), Wh2,
                      bh2.reshape(1, D), Wh3.reshape(1, D),
                      jnp.broadcast_to(bh3[None, :], (1, D)))
    return pooled[:, :1]
